# baseline (speedup 1.0000x reference)
import functools

import jax
import jax.numpy as jnp
from jax import lax
from jax.experimental import pallas as pl
from jax.experimental.pallas import tpu as pltpu
from jax.experimental.pallas import tpu_sc as plsc

NC, NS, L = 2, 16, 16
NW = NC * NS
B, N, C1 = 16, 512, 81
ROWS = B * N
OROW = 4 * C1
LROW = C1
RW = ROWS // NW
CHUNK = 64
NCHUNK = RW // CHUNK
GV = (LROW - 1) // L

_mesh = plsc.VectorSubcoreMesh(core_axis_name="c", subcore_axis_name="s")


@functools.partial(
    pl.kernel,
    out_type=(
        jax.ShapeDtypeStruct((NW, L), jnp.float32),
        jax.ShapeDtypeStruct((NW, L), jnp.float32),
    ),
    mesh=_mesh,
    compiler_params=pltpu.CompilerParams(
        use_tc_tiling_on_sc=False, needs_layout_passes=False
    ),
    scratch_types=[
        pltpu.VMEM((CHUNK * OROW,), jnp.float32),
        pltpu.VMEM((CHUNK * OROW,), jnp.float32),
        pltpu.VMEM((CHUNK * OROW,), jnp.float32),
        pltpu.VMEM((CHUNK * OROW,), jnp.float32),
        pltpu.VMEM((CHUNK * LROW,), jnp.float32),
        pltpu.VMEM((CHUNK * LROW,), jnp.float32),
        pltpu.VMEM((L,), jnp.float32),
        pltpu.SemaphoreType.DMA,
        pltpu.SemaphoreType.DMA,
    ],
)
def _sc_loss(o_hbm, t_hbm, l_hbm, num_hbm, den_hbm,
             o0, o1, t0, t1, l0, l1, stage, sem0, sem1):
    wid = lax.axis_index("s") * NC + lax.axis_index("c")
    row0 = wid * RW
    bufs = ((o0, t0, l0), (o1, t1, l1))
    sems = (sem0, sem1)

    def start(ci):
        slot = ci % 2
        r = row0 + ci * CHUNK
        ob, tb, lb = bufs[slot]
        return (
            pltpu.async_copy(o_hbm.at[pl.ds(r * OROW, CHUNK * OROW)], ob, sems[slot]),
            pltpu.async_copy(t_hbm.at[pl.ds(r * OROW, CHUNK * OROW)], tb, sems[slot]),
            pltpu.async_copy(l_hbm.at[pl.ds(r * LROW, CHUNK * LROW)], lb, sems[slot]),
        )

    lanes4 = lax.iota(jnp.int32, L) * 4
    num_acc = jnp.zeros((L,), jnp.float32)
    den_acc = jnp.zeros((L,), jnp.float32)

    descs = start(0)
    for ci in range(NCHUNK):
        slot = ci % 2
        if ci + 1 < NCHUNK:
            nxt = start(ci + 1)
        for d in descs:
            d.wait()

        o_ref, t_ref, l_ref = bufs[slot]

        def row_body(r, carry):
            num, den = carry
            rbase = jnp.full((L,), r * OROW, jnp.int32)
            for g in range(GV):
                lab = l_ref[pl.ds(r * LROW + 1 + g * L, L)]
                m = lab == 1.0
                den = den + lab
                gacc = jnp.zeros((L,), jnp.float32)
                for j in range(4):
                    idx = rbase + (lanes4 + (4 + 64 * g + j))
                    o = plsc.load_gather(o_ref, [idx])
                    t = plsc.load_gather(t_ref, [idx])
                    d = o - t
                    ad = jnp.abs(d)
                    f = jnp.where(ad < 1.0, 0.5 * (d * d), ad - 0.5)
                    gacc = gacc + f
                num = num + jnp.where(m, gacc, 0.0)
            return num, den

        num_acc, den_acc = lax.fori_loop(0, CHUNK, row_body, (num_acc, den_acc))
        if ci + 1 < NCHUNK:
            descs = nxt

    stage[...] = num_acc
    pltpu.sync_copy(stage, num_hbm.at[wid])
    stage[...] = den_acc
    pltpu.sync_copy(stage, den_hbm.at[wid])


def kernel(output, target, labels_target):
    o = output.reshape(ROWS * OROW)
    t = target.reshape(ROWS * OROW)
    lt = labels_target.reshape(ROWS * LROW)
    num, den = _sc_loss(o, t, lt)
    b = jnp.sum(den) + jnp.float32(0.0001 * ROWS * (C1 - 1))
    return jnp.sum(num) / b


# speedup vs baseline: 1.3166x; 1.3166x over previous
import functools

import jax
import jax.numpy as jnp
from jax import lax
from jax.experimental import pallas as pl
from jax.experimental.pallas import tpu as pltpu
from jax.experimental.pallas import tpu_sc as plsc

NC, NS, L = 2, 16, 16
NW = NC * NS
B, N, C1 = 16, 512, 81
ROWS = B * N
OROW = 4 * C1
LROW = C1
RW = ROWS // NW
CHUNK = 64
NCHUNK = RW // CHUNK

_mesh = plsc.VectorSubcoreMesh(core_axis_name="c", subcore_axis_name="s")


def _expand4(lab, base):
    perm = (lax.iota(jnp.int32, L) // 4) + base
    return lab.at[perm].get(mode="promise_in_bounds")


@functools.partial(
    pl.kernel,
    out_type=(
        jax.ShapeDtypeStruct((NW, L), jnp.float32),
        jax.ShapeDtypeStruct((NW, L), jnp.float32),
    ),
    mesh=_mesh,
    compiler_params=pltpu.CompilerParams(
        use_tc_tiling_on_sc=True, needs_layout_passes=False
    ),
    scratch_types=[
        pltpu.VMEM((CHUNK, OROW), jnp.float32),
        pltpu.VMEM((CHUNK, OROW), jnp.float32),
        pltpu.VMEM((CHUNK, OROW), jnp.float32),
        pltpu.VMEM((CHUNK, OROW), jnp.float32),
        pltpu.VMEM((CHUNK, LROW), jnp.float32),
        pltpu.VMEM((CHUNK, LROW), jnp.float32),
        pltpu.VMEM((L,), jnp.float32),
        pltpu.SemaphoreType.DMA,
        pltpu.SemaphoreType.DMA,
    ],
)
def _sc_loss(o_hbm, t_hbm, l_hbm, num_hbm, den_hbm,
             o0, o1, t0, t1, l0, l1, stage, sem0, sem1):
    wid = lax.axis_index("s") * NC + lax.axis_index("c")
    b = wid // 2
    n0 = (wid % 2) * RW
    bufs = ((o0, t0, l0), (o1, t1, l1))
    sems = (sem0, sem1)

    def start(ci):
        slot = ci % 2
        n = n0 + ci * CHUNK
        ob, tb, lb = bufs[slot]
        return (
            pltpu.async_copy(o_hbm.at[b, pl.ds(n, CHUNK), :], ob, sems[slot]),
            pltpu.async_copy(t_hbm.at[b, pl.ds(n, CHUNK), :], tb, sems[slot]),
            pltpu.async_copy(l_hbm.at[b, pl.ds(n, CHUNK), :], lb, sems[slot]),
        )

    lane = lax.iota(jnp.int32, L)
    head_valid = lane >= 4
    tail_valid = lane >= 12
    den_head = lane >= 1
    den_tail = lane == 15

    num_acc = jnp.zeros((L,), jnp.float32)
    den_acc = jnp.zeros((L,), jnp.float32)

    descs = start(0)
    for ci in range(NCHUNK):
        slot = ci % 2
        if ci + 1 < NCHUNK:
            nxt = start(ci + 1)
        for d in descs:
            d.wait()

        o_ref, t_ref, l_ref = bufs[slot]

        def row_body(r, carry):
            num, den = carry
            labs = [l_ref[r, pl.ds(16 * w, L)] for w in range(5)]
            labx = l_ref[r, pl.ds(65, L)]
            den = den + jnp.where(den_head, labs[0], 0.0)
            den = den + labs[1] + labs[2] + labs[3] + labs[4]
            den = den + jnp.where(den_tail, labx, 0.0)

            def huber(c0):
                o = o_ref[r, pl.ds(c0, L)]
                t = t_ref[r, pl.ds(c0, L)]
                d = o - t
                ad = jnp.abs(d)
                mn = jnp.minimum(ad, 1.0)
                return mn * (ad - 0.5 * mn)

            for u in range(20):
                mv = _expand4(labs[u // 4], (u % 4) * 4) == 1.0
                if u == 0:
                    mv = mv & head_valid
                num = num + jnp.where(mv, huber(16 * u), 0.0)
            mv = (_expand4(labx, 12) == 1.0) & tail_valid
            num = num + jnp.where(mv, huber(308), 0.0)
            return num, den

        num_acc, den_acc = lax.fori_loop(0, CHUNK, row_body, (num_acc, den_acc))
        if ci + 1 < NCHUNK:
            descs = nxt

    stage[...] = num_acc
    pltpu.sync_copy(stage, num_hbm.at[wid])
    stage[...] = den_acc
    pltpu.sync_copy(stage, den_hbm.at[wid])


def kernel(output, target, labels_target):
    num, den = _sc_loss(output, target, labels_target)
    b = jnp.sum(den) + jnp.float32(0.0001 * ROWS * (C1 - 1))
    return jnp.sum(num) / b
